# baseline (speedup 1.0000x reference)
import functools

import jax
import jax.numpy as jnp
from jax import lax
from jax.experimental import pallas as pl
from jax.experimental.pallas import tpu as pltpu
from jax.experimental.pallas import tpu_sc as plsc

N_VERTS = 35709
ROW = N_VERTS * 3
BATCH = 128
NUM_WORKERS = 32
ROWS_PER_W = BATCH // NUM_WORKERS

REST, EYE, MOUTH = 0, 1, 2
N_MOUTH = 1700 * 3 * BATCH
N_EYE = 1600 * 3 * BATCH
N_REST = (N_VERTS - 3300) * 3 * BATCH

CH = 12288

def _chunk_list():
    segs = [(0, 12000, REST), (12000, 2400, EYE), (14400, 9600, REST),
            (24000, 2400, EYE), (26400, 12000, REST), (38400, 5088, MOUTH)]
    out = []
    for s, l, k in segs:
        while l > CH:
            out.append((s, CH, k)); s += CH; l -= CH
        out.append((s, l, k))
    out.append((43488, 16, "SPLIT"))
    s, l = 43504, 63616
    while l > CH:
        out.append((s, CH, REST)); s += CH; l -= CH
    out.append((s, l, REST))
    out.append((107112, 15, "TAIL"))
    return tuple(out)

CHUNKS = _chunk_list()

@functools.cache
def _build_sc_kernel():
    mesh = plsc.VectorSubcoreMesh(core_axis_name="c", subcore_axis_name="s")
    return functools.partial(
        pl.kernel,
        mesh=mesh,
        out_type=jax.ShapeDtypeStruct((NUM_WORKERS, 3, 16), jnp.float32),
        scratch_types=[
            pltpu.VMEM((CH,), jnp.float32),
            pltpu.VMEM((CH,), jnp.float32),
            pltpu.VMEM((3, 16), jnp.float32),
        ],
        compiler_params=pltpu.CompilerParams(use_tc_tiling_on_sc=False),
    )(_region_l1_sc)


def _region_l1_sc(x_hbm, y_hbm, out_hbm, xbuf, ybuf, accbuf):
    wid = lax.axis_index("s") * 2 + lax.axis_index("c")
    zero = jnp.zeros((16,), jnp.float32)
    lane = lax.iota(jnp.int32, 16)

    def row_body(r, accs):
        row = wid * ROWS_PER_W + r
        acc = list(accs)
        for s, l, kind in CHUNKS:
            pltpu.sync_copy(x_hbm.at[row, pl.ds(s, l)], xbuf.at[pl.ds(0, l)])
            pltpu.sync_copy(y_hbm.at[row, pl.ds(s, l)], ybuf.at[pl.ds(0, l)])
            if kind == "SPLIT":
                d = jnp.abs(xbuf[pl.ds(0, 16)] - ybuf[pl.ds(0, 16)])
                acc[MOUTH] = acc[MOUTH] + jnp.where(lane < 12, d, 0.0)
                acc[REST] = acc[REST] + jnp.where(lane >= 12, d, 0.0)
            elif kind == "TAIL":
                d = jnp.abs(xbuf[pl.ds(0, 16)] - ybuf[pl.ds(0, 16)])
                acc[REST] = acc[REST] + jnp.where(
                    (lane >= 8) & (lane < 15), d, 0.0)
            else:
                def vbody(i, a):
                    xv = xbuf[pl.ds(i * 16, 16)]
                    yv = ybuf[pl.ds(i * 16, 16)]
                    return a + jnp.abs(xv - yv)
                part = lax.fori_loop(0, l // 16, vbody, zero)
                acc[kind] = acc[kind] + part
        return tuple(acc)

    acc_rest, acc_eye, acc_mouth = lax.fori_loop(
        0, ROWS_PER_W, row_body, (zero, zero, zero))
    accbuf[0, :] = acc_rest
    accbuf[1, :] = acc_eye
    accbuf[2, :] = acc_mouth
    pltpu.sync_copy(accbuf, out_hbm.at[wid])


def kernel(input, target):
    x = input.reshape(BATCH, ROW)
    y = target.reshape(BATCH, ROW)
    partials = _build_sc_kernel()(x, y)
    sums = partials.sum(axis=(0, 2))
    mouth_loss = sums[MOUTH] / N_MOUTH
    eye_loss = sums[EYE] / N_EYE
    rest_loss = sums[REST] / N_REST
    return (mouth_loss, eye_loss, rest_loss)


# speedup vs baseline: 3.5641x; 3.5641x over previous
import functools

import jax
import jax.numpy as jnp
from jax import lax
from jax.experimental import pallas as pl
from jax.experimental.pallas import tpu as pltpu
from jax.experimental.pallas import tpu_sc as plsc

N_VERTS = 35709
ROW = N_VERTS * 3
ROW_PAD = 107136
BATCH = 128
NUM_WORKERS = 32
ROWS_PER_W = BATCH // NUM_WORKERS

REST, EYE, MOUTH, SPLIT = 0, 1, 2, 3
N_MOUTH = 1700 * 3 * BATCH
N_EYE = 1600 * 3 * BATCH
N_REST = (N_VERTS - 3300) * 3 * BATCH

CH = 24576

_SEGS = ((0, 12000, REST), (12000, 14400, EYE), (14400, 24000, REST),
         (24000, 26400, EYE), (26400, 38400, REST), (38400, 43488, MOUTH),
         (43488, 43504, SPLIT), (43504, ROW_PAD, REST))


def _chunk_table():
    chunks = []
    cs = 0
    while cs < ROW_PAD:
        ce = min(cs + CH, ROW_PAD)
        spans = []
        for s, e, kind in _SEGS:
            lo, hi = max(s, cs), min(e, ce)
            if lo < hi:
                spans.append((lo - cs, hi - lo, kind))
        chunks.append((cs, ce - cs, tuple(spans)))
        cs = ce
    return tuple(chunks)

DMA_CHUNKS = _chunk_table()
UNROLL = 8


def _span_sum(xb, yb, off, nvec):
    a0 = jnp.zeros((16,), jnp.float32)
    a1 = jnp.zeros((16,), jnp.float32)
    n_u = nvec // UNROLL

    if n_u > 0:
        def body(i, accs):
            b0, b1 = accs
            base = off + i * (16 * UNROLL)
            for u in range(UNROLL):
                o = base + u * 16
                v = jnp.abs(xb[pl.ds(o, 16)] - yb[pl.ds(o, 16)])
                if u % 2 == 0:
                    b0 = b0 + v
                else:
                    b1 = b1 + v
            return (b0, b1)
        a0, a1 = lax.fori_loop(0, n_u, body, (a0, a1))
    base = off + n_u * (16 * UNROLL)
    for u in range(nvec % UNROLL):
        o = base + u * 16
        v = jnp.abs(xb[pl.ds(o, 16)] - yb[pl.ds(o, 16)])
        if u % 2 == 0:
            a0 = a0 + v
        else:
            a1 = a1 + v
    return a0 + a1


def _region_l1_sc(x_hbm, y_hbm, out_hbm, xbuf0, xbuf1, ybuf0, ybuf1,
                  accbuf, sem0, sem1):
    wid = lax.axis_index("s") * 2 + lax.axis_index("c")
    zero = jnp.zeros((16,), jnp.float32)
    lane = lax.iota(jnp.int32, 16)
    sems = (sem0, sem1)
    xbufs = (xbuf0, xbuf1)
    ybufs = (ybuf0, ybuf1)
    nchunks = len(DMA_CHUNKS)

    def issue(row, c, slot):
        cs, cl, _ = DMA_CHUNKS[c]
        hx = pltpu.async_copy(x_hbm.at[row, pl.ds(cs, cl)],
                              xbufs[slot].at[pl.ds(0, cl)], sems[slot])
        hy = pltpu.async_copy(y_hbm.at[row, pl.ds(cs, cl)],
                              ybufs[slot].at[pl.ds(0, cl)], sems[slot])
        return hx, hy

    def row_body(r, accs):
        row = wid * ROWS_PER_W + r
        acc = list(accs)
        handles = [None, None]
        handles[0] = issue(row, 0, 0)
        handles[1] = issue(row, 1, 1)
        for c, (cs, cl, spans) in enumerate(DMA_CHUNKS):
            slot = c % 2
            hx, hy = handles[slot]
            hx.wait()
            hy.wait()
            xb, yb = xbufs[slot], ybufs[slot]
            for off, ln, kind in spans:
                if kind == SPLIT:
                    d = jnp.abs(xb[pl.ds(off, 16)] - yb[pl.ds(off, 16)])
                    acc[MOUTH] = acc[MOUTH] + jnp.where(lane < 12, d, 0.0)
                    acc[REST] = acc[REST] + jnp.where(lane >= 12, d, 0.0)
                else:
                    acc[kind] = acc[kind] + _span_sum(xb, yb, off, ln // 16)
            if c + 2 < nchunks:
                handles[slot] = issue(row, c + 2, slot)
        return tuple(acc)

    acc_rest, acc_eye, acc_mouth = lax.fori_loop(
        0, ROWS_PER_W, row_body, (zero, zero, zero))
    accbuf[0, :] = acc_rest
    accbuf[1, :] = acc_eye
    accbuf[2, :] = acc_mouth
    pltpu.sync_copy(accbuf, out_hbm.at[wid])


@functools.cache
def _build_sc_kernel():
    mesh = plsc.VectorSubcoreMesh(core_axis_name="c", subcore_axis_name="s")
    return functools.partial(
        pl.kernel,
        mesh=mesh,
        out_type=jax.ShapeDtypeStruct((NUM_WORKERS, 3, 16), jnp.float32),
        scratch_types=[
            pltpu.VMEM((CH,), jnp.float32),
            pltpu.VMEM((CH,), jnp.float32),
            pltpu.VMEM((CH,), jnp.float32),
            pltpu.VMEM((CH,), jnp.float32),
            pltpu.VMEM((3, 16), jnp.float32),
            pltpu.SemaphoreType.DMA,
            pltpu.SemaphoreType.DMA,
        ],
        compiler_params=pltpu.CompilerParams(use_tc_tiling_on_sc=False),
    )(_region_l1_sc)


def kernel(input, target):
    x = input.reshape(BATCH, ROW)
    y = target.reshape(BATCH, ROW)
    pad = ROW_PAD - ROW
    x = jnp.pad(x, ((0, 0), (0, pad)))
    y = jnp.pad(y, ((0, 0), (0, pad)))
    partials = _build_sc_kernel()(x, y)
    sums = partials.sum(axis=(0, 2))
    mouth_loss = sums[MOUTH] / N_MOUTH
    eye_loss = sums[EYE] / N_EYE
    rest_loss = sums[REST] / N_REST
    return (mouth_loss, eye_loss, rest_loss)
